# baseline (speedup 1.0000x reference)
import jax
import jax.numpy as jnp
from jax.experimental import pallas as pl
from jax.experimental.pallas import tpu as pltpu

GRID_XYZ = (100, 100, 8)
SPLIT3 = (4, 4, 4)
CDIM = 128


def _bn_stats_affine(x, p):
    m = jnp.mean(x, axis=0)
    v = jnp.var(x, axis=0)
    a = p["g"] / jnp.sqrt(v + 1e-5)
    c = p["b"] - m * a
    return a, c


def _proc_body(u3_ref, cam_ref, w4_ref, b4_ref, wc_ref, bc_ref, out_ref):
    h4 = jax.lax.dot_general(u3_ref[...], w4_ref[...], (((1,), (0,)), ((), ())),
                             preferred_element_type=jnp.float32)
    cr = jax.lax.dot_general(cam_ref[...], wc_ref[...], (((1,), (0,)), ((), ())),
                             preferred_element_type=jnp.float32)
    out_ref[...] = h4 + b4_ref[...] + cr + bc_ref[...]


def _proc_pallas(u3, cam, params):
    M = u3.shape[0]
    BLK = 2048
    grid = (M // BLK,)
    w4, b4 = params["fc4"]["w"], params["fc4"]["b"].reshape(1, CDIM)
    wc, bc = params["cam"]["w"], params["cam"]["b"].reshape(1, CDIM)
    return pl.pallas_call(
        _proc_body,
        grid=grid,
        in_specs=[
            pl.BlockSpec((BLK, u3.shape[1]), lambda i: (i, 0)),
            pl.BlockSpec((BLK, cam.shape[1]), lambda i: (i, 0)),
            pl.BlockSpec(w4.shape, lambda i: (0, 0)),
            pl.BlockSpec(b4.shape, lambda i: (0, 0)),
            pl.BlockSpec(wc.shape, lambda i: (0, 0)),
            pl.BlockSpec(bc.shape, lambda i: (0, 0)),
        ],
        out_specs=pl.BlockSpec((BLK, CDIM), lambda i: (i, 0)),
        out_shape=jax.ShapeDtypeStruct((M, CDIM), jnp.float32),
    )(u3, cam, w4, b4, wc, bc)


def kernel(points, grid_ind, cam_point_features, params):
    X, Y, Z = GRID_XYZ
    NEG = -1e30
    B, N = points.shape[0], points.shape[1]
    fea = points[..., :5].reshape(B * N, 5)
    ind = grid_ind.reshape(B * N, 3)
    camf = cam_point_features.reshape(B * N, cam_point_features.shape[-1])
    bidx = jnp.repeat(jnp.arange(B), N)

    a0, c0 = _bn_stats_affine(fea, params["bn0"])
    h = fea * a0 + c0
    h = h @ params["fc1"]["w"] + params["fc1"]["b"]
    a1, c1 = _bn_stats_affine(h, params["bn1"])
    h = jax.nn.relu(h * a1 + c1)
    h = h @ params["fc2"]["w"] + params["fc2"]["b"]
    a2, c2 = _bn_stats_affine(h, params["bn2"])
    h = jax.nn.relu(h * a2 + c2)
    h = h @ params["fc3"]["w"] + params["fc3"]["b"]
    a3, c3 = _bn_stats_affine(h, params["bn3"])
    u3 = jax.nn.relu(h * a3 + c3)

    proc = _proc_pallas(u3, camf, params)

    key = ((bidx * X + ind[:, 0]) * Y + ind[:, 1]) * Z + ind[:, 2]
    dense = jax.ops.segment_max(proc, key, num_segments=B * X * Y * Z)
    dense = jnp.where(jnp.isneginf(dense), NEG, dense)
    dense = dense.reshape(B, X, Y, Z, CDIM)

    def unpool(p):
        return jnp.where(p < -1e29, 0.0, p)

    kz = Z // SPLIT3[2]
    kx = X // SPLIT3[0]
    ky = Y // SPLIT3[1]
    pxy = unpool(dense.reshape(B, X, Y, SPLIT3[2], kz, CDIM).max(axis=4))
    fxy = pxy.reshape(B, X, Y, SPLIT3[2] * CDIM)
    txy = jax.nn.relu(fxy @ params["xy1"]["w"] + params["xy1"]["b"])
    txy = txy @ params["xy2"]["w"] + params["xy2"]["b"]
    txy = jnp.transpose(txy, (0, 3, 1, 2))

    pyz = unpool(dense.reshape(B, SPLIT3[0], kx, Y, Z, CDIM).max(axis=2))
    pyz = jnp.transpose(pyz, (0, 2, 3, 1, 4))
    fyz = pyz.reshape(B, Y, Z, SPLIT3[0] * CDIM)
    tyz = jax.nn.relu(fyz @ params["yz1"]["w"] + params["yz1"]["b"])
    tyz = tyz @ params["yz2"]["w"] + params["yz2"]["b"]
    tyz = jnp.transpose(tyz, (0, 3, 1, 2))

    pxz = unpool(dense.reshape(B, X, SPLIT3[1], ky, Z, CDIM).max(axis=3))
    pxz = jnp.transpose(pxz, (0, 1, 3, 2, 4))
    fxz = pxz.reshape(B, X, Z, SPLIT3[1] * CDIM)
    txz = jax.nn.relu(fxz @ params["xz1"]["w"] + params["xz1"]["b"])
    txz = txz @ params["xz2"]["w"] + params["xz2"]["b"]
    txz = jnp.transpose(txz, (0, 3, 1, 2))
    return (txy, tyz, txz)


# speedup vs baseline: 1.0389x; 1.0389x over previous
import functools

import jax
import jax.numpy as jnp
from jax.experimental import pallas as pl
from jax.experimental.pallas import tpu as pltpu

GRID_XYZ = (100, 100, 8)
SPLIT3 = (4, 4, 4)
CDIM = 128
NEG = -1e30


def _moments_body(x_ref, s_ref, m_ref, acc_s, acc_m):
    i = pl.program_id(0)

    @pl.when(i == 0)
    def _init():
        acc_s[...] = jnp.zeros_like(acc_s)
        acc_m[...] = jnp.zeros_like(acc_m)

    x = x_ref[...]
    acc_s[...] += jnp.sum(x, axis=0, keepdims=True)
    acc_m[...] += jax.lax.dot_general(x, x, (((0,), (0,)), ((), ())),
                                      preferred_element_type=jnp.float32)

    @pl.when(i == pl.num_programs(0) - 1)
    def _emit():
        s_ref[...] = acc_s[...]
        m_ref[...] = acc_m[...]


def _moments(x, blk):
    n, d = x.shape
    return pl.pallas_call(
        _moments_body,
        grid=(n // blk,),
        in_specs=[pl.BlockSpec((blk, d), lambda i: (i, 0))],
        out_specs=[pl.BlockSpec((1, d), lambda i: (0, 0)),
                   pl.BlockSpec((d, d), lambda i: (0, 0))],
        out_shape=[jax.ShapeDtypeStruct((1, d), jnp.float32),
                   jax.ShapeDtypeStruct((d, d), jnp.float32)],
        scratch_shapes=[pltpu.VMEM((1, d), jnp.float32),
                        pltpu.VMEM((d, d), jnp.float32)],
    )(x)


def _accum_moments(u, i, s_ref, m_ref, acc_s, acc_m):
    @pl.when(i == 0)
    def _init():
        acc_s[...] = jnp.zeros_like(acc_s)
        acc_m[...] = jnp.zeros_like(acc_m)

    acc_s[...] += jnp.sum(u, axis=0, keepdims=True)
    acc_m[...] += jax.lax.dot_general(u, u, (((0,), (0,)), ((), ())),
                                      preferred_element_type=jnp.float32)

    @pl.when(i == pl.num_programs(0) - 1)
    def _emit():
        s_ref[...] = acc_s[...]
        m_ref[...] = acc_m[...]


def _mm(a, w):
    return jax.lax.dot_general(a, w, (((1,), (0,)), ((), ())),
                               preferred_element_type=jnp.float32)


def _mlp1_moments_body(x_ref, w1_ref, b1_ref, s_ref, m_ref, acc_s, acc_m):
    u = jax.nn.relu(_mm(x_ref[...], w1_ref[...]) + b1_ref[...])
    _accum_moments(u, pl.program_id(0), s_ref, m_ref, acc_s, acc_m)


def _mlp2_moments_body(x_ref, w1_ref, b1_ref, w2_ref, b2_ref,
                       s_ref, m_ref, acc_s, acc_m):
    u = jax.nn.relu(_mm(x_ref[...], w1_ref[...]) + b1_ref[...])
    u = jax.nn.relu(_mm(u, w2_ref[...]) + b2_ref[...])
    _accum_moments(u, pl.program_id(0), s_ref, m_ref, acc_s, acc_m)


def _mlp_moments(x, w1, b1, w2, b2, blk):
    n = x.shape[0]
    d = w1.shape[1] if w2 is None else w2.shape[1]
    args = [x, w1, b1] + ([] if w2 is None else [w2, b2])
    body = _mlp1_moments_body if w2 is None else _mlp2_moments_body
    return pl.pallas_call(
        body,
        grid=(n // blk,),
        in_specs=[pl.BlockSpec((blk, x.shape[1]), lambda i: (i, 0))]
        + [pl.BlockSpec(a.shape, lambda i: (0, 0)) for a in args[1:]],
        out_specs=[pl.BlockSpec((1, d), lambda i: (0, 0)),
                   pl.BlockSpec((d, d), lambda i: (0, 0))],
        out_shape=[jax.ShapeDtypeStruct((1, d), jnp.float32),
                   jax.ShapeDtypeStruct((d, d), jnp.float32)],
        scratch_shapes=[pltpu.VMEM((1, d), jnp.float32),
                        pltpu.VMEM((d, d), jnp.float32)],
    )(*args)


def _proc_body(x_ref, cam_ref, w1_ref, b1_ref, w2_ref, b2_ref, w3_ref, b3_ref,
               w4_ref, b4_ref, wc_ref, bc_ref, out_ref):
    def mm(a, w):
        return jax.lax.dot_general(a, w, (((1,), (0,)), ((), ())),
                                   preferred_element_type=jnp.float32)

    u = jax.nn.relu(mm(x_ref[...], w1_ref[...]) + b1_ref[...])
    u = jax.nn.relu(mm(u, w2_ref[...]) + b2_ref[...])
    u = jax.nn.relu(mm(u, w3_ref[...]) + b3_ref[...])
    out_ref[...] = (mm(u, w4_ref[...]) + b4_ref[...]
                    + mm(cam_ref[...], wc_ref[...]) + bc_ref[...])


def _proc_pallas(x, cam, weights, blk):
    n = x.shape[0]
    return pl.pallas_call(
        _proc_body,
        grid=(n // blk,),
        in_specs=[pl.BlockSpec((blk, x.shape[1]), lambda i: (i, 0)),
                  pl.BlockSpec((blk, cam.shape[1]), lambda i: (i, 0))]
        + [pl.BlockSpec(w.shape, lambda i: (0, 0)) for w in weights],
        out_specs=pl.BlockSpec((blk, CDIM), lambda i: (i, 0)),
        out_shape=jax.ShapeDtypeStruct((n, CDIM), jnp.float32),
    )(x, cam, *weights)


def _plane_mlp_body(x_ref, w1_ref, b1_ref, w2_ref, b2_ref, out_ref):
    def mm(a, w):
        return jax.lax.dot_general(a, w, (((1,), (0,)), ((), ())),
                                   preferred_element_type=jnp.float32)

    x = x_ref[...]
    x = jnp.where(x < -1e29, 0.0, x)
    u = jax.nn.relu(mm(x, w1_ref[...]) + b1_ref[...])
    out_ref[...] = mm(u, w2_ref[...]) + b2_ref[...]


def _plane_mlp(x, w1, b1, w2, b2, blk):
    r = x.shape[0]
    return pl.pallas_call(
        _plane_mlp_body,
        grid=(r // blk,),
        in_specs=[pl.BlockSpec((blk, x.shape[1]), lambda i: (i, 0)),
                  pl.BlockSpec(w1.shape, lambda i: (0, 0)),
                  pl.BlockSpec(b1.shape, lambda i: (0, 0)),
                  pl.BlockSpec(w2.shape, lambda i: (0, 0)),
                  pl.BlockSpec(b2.shape, lambda i: (0, 0))],
        out_specs=pl.BlockSpec((blk, CDIM), lambda i: (i, 0)),
        out_shape=jax.ShapeDtypeStruct((r, CDIM), jnp.float32),
    )(x, w1, b1, w2, b2)


def _fold_next(mu, cov, a, c, w, b, p_next):
    mean_h = mu @ w + b
    var_h = jnp.sum(w * (cov @ w), axis=0)
    a2 = p_next["g"] / jnp.sqrt(var_h + 1e-5)
    c2 = p_next["b"] - mean_h * a2
    return a2, c2


def kernel(points, grid_ind, cam_point_features, params):
    X, Y, Z = GRID_XYZ
    B, N = points.shape[0], points.shape[1]
    M = B * N
    fea = points[..., :5].reshape(M, 5)
    feap = jnp.pad(fea, ((0, 0), (0, 3)))
    ind = grid_ind.reshape(M, 3)
    camf = cam_point_features.reshape(M, cam_point_features.shape[-1])
    bidx = jnp.repeat(jnp.arange(B), N)

    s0, m0 = _moments(feap, 8192)
    mu0 = s0[0] / M
    cov0 = m0 / M - jnp.outer(mu0, mu0)
    g0 = jnp.pad(params["bn0"]["g"], (0, 3))
    bb0 = jnp.pad(params["bn0"]["b"], (0, 3))
    var0 = jnp.diagonal(cov0)
    a0 = jnp.where(jnp.arange(8) < 5, g0 / jnp.sqrt(var0 + 1e-5), 0.0)
    c0 = bb0 - mu0 * a0

    w1 = jnp.pad(params["fc1"]["w"], ((0, 3), (0, 0)))
    w1f = a0[:, None] * w1
    b1f = c0 @ w1 + params["fc1"]["b"]
    mean1 = mu0 @ w1f + b1f
    var1 = jnp.sum(w1f * (cov0 @ w1f), axis=0)
    a1 = params["bn1"]["g"] / jnp.sqrt(var1 + 1e-5)
    c1 = params["bn1"]["b"] - mean1 * a1
    w1ff = w1f * a1[None, :]
    b1ff = b1f * a1 + c1

    s1, m1 = _mlp_moments(feap, w1ff, b1ff.reshape(1, -1), None, None, 8192)
    mu1 = s1[0] / M
    cov1 = m1 / M - jnp.outer(mu1, mu1)
    w2, b2 = params["fc2"]["w"], params["fc2"]["b"]
    a2, c2 = _fold_next(mu1, cov1, None, None, w2, b2, params["bn2"])
    w2ff = w2 * a2[None, :]
    b2ff = b2 * a2 + c2

    s2, m2 = _mlp_moments(feap, w1ff, b1ff.reshape(1, -1),
                          w2ff, b2ff.reshape(1, -1), 8192)
    mu2 = s2[0] / M
    cov2 = m2 / M - jnp.outer(mu2, mu2)
    w3, b3 = params["fc3"]["w"], params["fc3"]["b"]
    a3, c3 = _fold_next(mu2, cov2, None, None, w3, b3, params["bn3"])
    w3ff = w3 * a3[None, :]
    b3ff = b3 * a3 + c3

    weights = [w1ff, b1ff.reshape(1, -1), w2ff, b2ff.reshape(1, -1),
               w3ff, b3ff.reshape(1, -1),
               params["fc4"]["w"], params["fc4"]["b"].reshape(1, CDIM),
               params["cam"]["w"], params["cam"]["b"].reshape(1, CDIM)]
    proc = _proc_pallas(feap, camf, weights, 2048)

    key = ((bidx * X + ind[:, 0]) * Y + ind[:, 1]) * Z + ind[:, 2]
    dense = jax.ops.segment_max(proc, key, num_segments=B * X * Y * Z)
    dense = jnp.where(jnp.isneginf(dense), NEG, dense)
    dense = dense.reshape(B, X, Y, Z, CDIM)

    kz = Z // SPLIT3[2]
    kx = X // SPLIT3[0]
    ky = Y // SPLIT3[1]
    pxy = dense.reshape(B, X, Y, SPLIT3[2], kz, CDIM).max(axis=4)
    fxy = pxy.reshape(B * X * Y, SPLIT3[2] * CDIM)
    txy = _plane_mlp(fxy, params["xy1"]["w"], params["xy1"]["b"].reshape(1, -1),
                     params["xy2"]["w"], params["xy2"]["b"].reshape(1, -1), 2000)
    txy = jnp.transpose(txy.reshape(B, X, Y, CDIM), (0, 3, 1, 2))

    pyz = dense.reshape(B, SPLIT3[0], kx, Y, Z, CDIM).max(axis=2)
    pyz = jnp.transpose(pyz, (0, 2, 3, 1, 4))
    fyz = pyz.reshape(B * Y * Z, SPLIT3[0] * CDIM)
    tyz = _plane_mlp(fyz, params["yz1"]["w"], params["yz1"]["b"].reshape(1, -1),
                     params["yz2"]["w"], params["yz2"]["b"].reshape(1, -1), 1600)
    tyz = jnp.transpose(tyz.reshape(B, Y, Z, CDIM), (0, 3, 1, 2))

    pxz = dense.reshape(B, X, SPLIT3[1], ky, Z, CDIM).max(axis=3)
    pxz = jnp.transpose(pxz, (0, 1, 3, 2, 4))
    fxz = pxz.reshape(B * X * Z, SPLIT3[1] * CDIM)
    txz = _plane_mlp(fxz, params["xz1"]["w"], params["xz1"]["b"].reshape(1, -1),
                     params["xz2"]["w"], params["xz2"]["b"].reshape(1, -1), 1600)
    txz = jnp.transpose(txz.reshape(B, X, Z, CDIM), (0, 3, 1, 2))
    return (txy, tyz, txz)
